# baseline (speedup 1.0000x reference)
import functools
import math

import jax
import jax.numpy as jnp
from jax import lax
from jax.experimental import pallas as pl
from jax.experimental.pallas import tpu as pltpu
from jax.experimental.pallas import tpu_sc as plsc

D_MODEL = 64
SCALE = math.sqrt(D_MODEL)

NUM_WORKERS = 32
BLK = 128
LANES = 16
PAIR = 2 * D_MODEL


def _make_kernel(seq, batch):
    n_blocks_total = seq * (batch // BLK)
    npw = n_blocks_total // NUM_WORKERS
    bch = batch // BLK
    mesh = plsc.VectorSubcoreMesh(core_axis_name="c", subcore_axis_name="s")

    @functools.partial(
        pl.kernel,
        out_type=jax.ShapeDtypeStruct((seq, D_MODEL, batch), jnp.float32),
        mesh=mesh,
        scratch_types=[
            pltpu.VMEM((2, BLK), jnp.int32),
            pltpu.VMEM((2, BLK), jnp.int32),
            pltpu.VMEM((2, BLK, PAIR), jnp.float32),
            pltpu.VMEM((2, D_MODEL, BLK), jnp.float32),
            pltpu.SemaphoreType.DMA,
            pltpu.SemaphoreType.DMA,
            pltpu.SemaphoreType.DMA,
        ],
        compiler_params=pltpu.CompilerParams(
            use_tc_tiling_on_sc=True, needs_layout_passes=False),
    )
    def emb(xp_hbm, xo_hbm, tab2_hbm, out_hbm, idx_v, off_v, pair_v, out_v,
            isem, gsem, ssem):
        cid = lax.axis_index("c")
        sid = lax.axis_index("s")
        wid = sid * 2 + cid

        def coords(t):
            blk = wid * npw + t
            return blk // bch, (blk % bch) * BLK

        def idx_start(t, q):
            s, b0 = coords(t)
            pltpu.async_copy(xp_hbm.at[s, pl.ds(b0, BLK)], idx_v.at[q], isem)
            pltpu.async_copy(xo_hbm.at[s, pl.ds(b0, BLK)], off_v.at[q], isem)

        def idx_wait(t, q):
            s, b0 = coords(t)
            pltpu.make_async_copy(
                xp_hbm.at[s, pl.ds(b0, BLK)], idx_v.at[q], isem).wait()
            pltpu.make_async_copy(
                xo_hbm.at[s, pl.ds(b0, BLK)], off_v.at[q], isem).wait()

        def gat_start(q):
            pltpu.async_copy(tab2_hbm.at[idx_v.at[q]], pair_v.at[q], gsem)

        def gat_wait(q):
            pltpu.make_async_copy(
                tab2_hbm.at[idx_v.at[q]], pair_v.at[q], gsem).wait()

        def st_start(t, q):
            s, b0 = coords(t)
            pltpu.async_copy(
                out_v.at[q], out_hbm.at[s, :, pl.ds(b0, BLK)], ssem)

        def st_wait(t, q):
            s, b0 = coords(t)
            pltpu.make_async_copy(
                out_v.at[q], out_hbm.at[s, :, pl.ds(b0, BLK)], ssem).wait()

        def compute(q):
            for r0 in range(0, BLK, LANES):
                rows = r0 + lax.iota(jnp.int32, LANES)
                offs = off_v[q, pl.ds(r0, LANES)]

                def d_body(d, c):
                    vals = plsc.load_gather(pair_v.at[q], [rows, offs + d])
                    out_v[q, d, pl.ds(r0, LANES)] = vals * SCALE
                    return c

                lax.fori_loop(0, D_MODEL, d_body, 0)

        idx_start(0, 0)
        idx_start(1, 1)
        idx_wait(0, 0)
        gat_start(0)

        def step(t, carry):
            q = t % 2
            gat_wait(q)
            @pl.when(t + 1 < npw)
            def _():
                idx_wait(t + 1, 1 - q)
                gat_start(1 - q)
            @pl.when(t >= 2)
            def _():
                st_wait(t - 2, q)
            compute(q)
            @pl.when(t + 2 < npw)
            def _():
                idx_start(t + 2, q)
            st_start(t, q)
            return carry

        lax.fori_loop(0, npw, step, 0)
        st_wait(npw - 2, 0)
        st_wait(npw - 1, 1)

    return emb


def kernel(x, table):
    b, s = x.shape
    xt = jnp.transpose(x)
    xp = jnp.right_shift(xt, 1)
    xo = (xt & 1) * D_MODEL
    tab2 = table.reshape(table.shape[0] // 2, PAIR)
    outp = _make_kernel(s, b)(xp, xo, tab2)
    return jnp.transpose(outp, (2, 0, 1))


# speedup vs baseline: 1.6409x; 1.6409x over previous
import functools
import math

import jax
import jax.numpy as jnp
from jax import lax
from jax.experimental import pallas as pl
from jax.experimental.pallas import tpu as pltpu
from jax.experimental.pallas import tpu_sc as plsc

D_MODEL = 64
SCALE = math.sqrt(D_MODEL)

NUM_WORKERS = 32
BLK = 128
LANES = 16
PAIR = 2 * D_MODEL
NBUF = 4


def _make_kernel(seq, batch):
    n_blocks_total = seq * (batch // BLK)
    npw = n_blocks_total // NUM_WORKERS
    bch = batch // BLK
    mesh = plsc.VectorSubcoreMesh(core_axis_name="c", subcore_axis_name="s")

    @functools.partial(
        pl.kernel,
        out_type=jax.ShapeDtypeStruct((seq, D_MODEL, batch), jnp.float32),
        mesh=mesh,
        scratch_types=[
            pltpu.VMEM((NBUF, BLK), jnp.int32),
            pltpu.VMEM((NBUF, BLK), jnp.int32),
            pltpu.VMEM((NBUF, BLK, PAIR), jnp.float32),
            pltpu.VMEM((NBUF, D_MODEL, BLK), jnp.float32),
            pltpu.SemaphoreType.DMA,
            pltpu.SemaphoreType.DMA,
            pltpu.SemaphoreType.DMA,
        ],
        compiler_params=pltpu.CompilerParams(
            use_tc_tiling_on_sc=True, needs_layout_passes=False),
    )
    def emb(xp_hbm, xo_hbm, tab2_hbm, out_hbm, idx_v, off_v, pair_v, out_v,
            isem, gsem, ssem):
        cid = lax.axis_index("c")
        sid = lax.axis_index("s")
        wid = sid * 2 + cid
        iota = lax.iota(jnp.int32, LANES)

        def coords(t):
            blk = wid * npw + t
            return blk // bch, (blk % bch) * BLK

        def idx_start(t, b):
            s, b0 = coords(t)
            pltpu.async_copy(xp_hbm.at[s, pl.ds(b0, BLK)], idx_v.at[b], isem)
            pltpu.async_copy(xo_hbm.at[s, pl.ds(b0, BLK)], off_v.at[b], isem)

        def idx_wait(t, b):
            s, b0 = coords(t)
            pltpu.make_async_copy(
                xp_hbm.at[s, pl.ds(b0, BLK)], idx_v.at[b], isem).wait()
            pltpu.make_async_copy(
                xo_hbm.at[s, pl.ds(b0, BLK)], off_v.at[b], isem).wait()

        def gat_start(b):
            pltpu.async_copy(tab2_hbm.at[idx_v.at[b]], pair_v.at[b], gsem)

        def gat_wait(b):
            pltpu.make_async_copy(
                tab2_hbm.at[idx_v.at[b]], pair_v.at[b], gsem).wait()

        def st_start(t, b):
            s, b0 = coords(t)
            pltpu.async_copy(
                out_v.at[b], out_hbm.at[s, :, pl.ds(b0, BLK)], ssem)

        def st_wait(t, b):
            s, b0 = coords(t)
            pltpu.make_async_copy(
                out_v.at[b], out_hbm.at[s, :, pl.ds(b0, BLK)], ssem).wait()

        def compute(b):
            def r_body(r8, c):
                r0 = r8 * LANES
                rows = r0 + iota
                offs = off_v[b, pl.ds(r0, LANES)]
                for d0 in range(0, D_MODEL, LANES):
                    for k in range(LANES):
                        dperm = d0 + ((iota + k) & (LANES - 1))
                        vals = plsc.load_gather(
                            pair_v.at[b], [rows, offs + dperm])
                        plsc.store_scatter(
                            out_v.at[b], [dperm, rows], vals * SCALE)
                return c

            lax.fori_loop(0, BLK // LANES, r_body, 0)

        for t in range(NBUF):
            idx_start(t, t)
        for t in range(NBUF - 1):
            idx_wait(t, t)
            gat_start(t)

        def step(t, carry):
            b = t % NBUF
            gat_wait(b)
            @pl.when(t + NBUF - 1 < npw)
            def _():
                bn = (t + NBUF - 1) % NBUF
                idx_wait(t + NBUF - 1, bn)
                gat_start(bn)
            @pl.when(t >= NBUF)
            def _():
                st_wait(t - NBUF, b)
            compute(b)
            @pl.when(t + NBUF < npw)
            def _():
                idx_start(t + NBUF, b)
            st_start(t, b)
            return carry

        lax.fori_loop(0, npw, step, 0)
        for t in range(npw - NBUF, npw):
            st_wait(t, t % NBUF)

    return emb


def kernel(x, table):
    b, s = x.shape
    xt = jnp.transpose(x)
    xp = jnp.right_shift(xt, 1)
    xo = (xt & 1) * D_MODEL
    tab2 = table.reshape(table.shape[0] // 2, PAIR)
    outp = _make_kernel(s, b)(xp, xo, tab2)
    return jnp.transpose(outp, (2, 0, 1))
